# baseline (speedup 1.0000x reference)
import functools

import jax
import jax.numpy as jnp
from jax import lax
from jax.experimental import pallas as pl
from jax.experimental.pallas import tpu as pltpu
from jax.experimental.pallas import tpu_sc as plsc

E = 8
K = 2
D = 768
DFF = 1024
N = 2048
TM = 128
NP = N * K
G = NP // TM + E
CAP = G * TM

NC = 2
NS = 16
NW = NC * NS
TPW = N // NW

_f32 = jnp.float32
_i32 = jnp.int32


_NBLK = NP // TM


def _gate_route_body(x_ref, wg_ref, pos_ref, te_ref, w_ref):
    xv = x_ref[...]
    logits = lax.dot_general(xv, wg_ref[...], (((1,), (1,)), ((), ())),
                             preferred_element_type=_f32)
    eio = lax.broadcasted_iota(_i32, (N, E), 1)
    m1 = jnp.max(logits, axis=1, keepdims=True)
    i1 = jnp.min(jnp.where(logits == m1, eio, E), axis=1, keepdims=True)
    masked = jnp.where(eio == i1, -jnp.inf, logits)
    m2 = jnp.max(masked, axis=1, keepdims=True)
    i2 = jnp.min(jnp.where(masked == m2, eio, E), axis=1, keepdims=True)
    w1 = jax.nn.sigmoid(m1 - m2)
    w2 = jax.nn.sigmoid(m2 - m1)
    w_ref[...] = jnp.broadcast_to(jnp.concatenate([w1, w2], axis=0), (NP, 16))

    e_all = jnp.concatenate([i1, i2], axis=0)
    oh_all = (e_all == lax.broadcasted_iota(_i32, (NP, E), 1)).astype(_f32)
    r = lax.broadcasted_iota(_i32, (TM, TM), 0)
    c = lax.broadcasted_iota(_i32, (TM, TM), 1)
    ltri = (c < r).astype(_f32)
    cnt = jnp.zeros((1, E), _f32)
    ranks = []
    for b in range(_NBLK):
        oh = lax.slice(oh_all, (b * TM, 0), ((b + 1) * TM, E))
        local = lax.dot_general(ltri, oh, (((1,), (0,)), ((), ())),
                                preferred_element_type=_f32)
        ranks.append(jnp.sum((local + cnt) * oh, axis=1, keepdims=True))
        cnt = cnt + jnp.sum(oh, axis=0, keepdims=True)
    rank_all = jnp.concatenate(ranks, axis=0)

    padded = jnp.floor((cnt + (TM - 1)) * (1.0 / TM)) * TM
    uio = lax.broadcasted_iota(_i32, (E, E), 0)
    cio = lax.broadcasted_iota(_i32, (E, E), 1)
    utri = (uio < cio).astype(_f32)
    start = lax.dot_general(padded, utri, (((1,), (0,)), ((), ())),
                            preferred_element_type=_f32)
    startsel = jnp.sum(start * oh_all, axis=1, keepdims=True)
    pos_ref[...] = (rank_all + startsel).astype(_i32)

    ivec = lax.broadcasted_iota(_i32, (1, 128), 1)
    te = jnp.full((1, 128), -1, _i32)
    enter = jnp.zeros((1, 128), _i32)
    for e in range(E):
        s_e = (start[0, e] * (1.0 / TM)).astype(_i32)
        act_e = (cnt[0, e] > 0.0).astype(_i32)
        te = te + (ivec >= s_e).astype(_i32)
        enter = enter + act_e * (ivec == s_e).astype(_i32)
    n_act = ((start[0, E - 1] + padded[0, E - 1]) * (1.0 / TM)).astype(_i32)
    te = jnp.where(ivec < n_act, te, E)

    rio = lax.broadcasted_iota(_i32, (128, 128), 0)
    cio2 = lax.broadcasted_iota(_i32, (128, 128), 1)
    uincl = (rio <= cio2).astype(_f32)
    rankp = lax.dot_general(enter.astype(_f32), uincl,
                            (((1,), (0,)), ((), ())),
                            preferred_element_type=_f32)
    sm1 = rankp - 1.0
    slot = (sm1 - jnp.floor(sm1 * (1.0 / 3.0)) * 3.0).astype(_i32)
    slot = jnp.maximum(slot, 0)

    s_pos, e_pos, v_pos = [], [], []
    entf = enter.astype(_f32)
    for p in range(E):
        m_p = entf * (rankp == (p + 1.0)).astype(_f32)
        s_pos.append(jnp.sum(m_p * ivec.astype(_f32)))
        e_pos.append(jnp.sum(m_p * te.astype(_f32)))
        v_pos.append(jnp.sum(m_p))
    fetch = jnp.full((1, 128), -1.0, _f32)
    for p in range(E - 2):
        hit = (ivec.astype(_f32) == s_pos[p]) * v_pos[p] * v_pos[p + 2]
        fetch = fetch + hit * (e_pos[p + 2] + 1.0)
    first3 = jnp.full((1, 128), -1.0, _f32)
    for p in range(3):
        first3 = first3 + (ivec == p).astype(_f32) * v_pos[p] * (e_pos[p] + 1.0)

    sched = jnp.concatenate(
        [te, enter, slot, fetch.astype(_i32), first3.astype(_i32),
         jnp.zeros((3, 128), _i32)], axis=0)
    te_ref[...] = sched


def _gate_route(xf, Wg):
    return pl.pallas_call(
        _gate_route_body,
        out_shape=(jax.ShapeDtypeStruct((NP, 1), _i32),
                   jax.ShapeDtypeStruct((8, 128), _i32),
                   jax.ShapeDtypeStruct((NP, 16), _f32)),
    )(xf, Wg)


@functools.cache
def _sc_kernels():
    mesh = plsc.VectorSubcoreMesh(core_axis_name="c", subcore_axis_name="s")

    @functools.partial(
        pl.kernel, mesh=mesh,
        out_type=jax.ShapeDtypeStruct((CAP, D), _f32),
        scratch_types=[
            pltpu.VMEM((TPW,), _i32),
            pltpu.VMEM((TPW,), _i32),
            pltpu.VMEM((TPW, D), _f32),
            pltpu.SemaphoreType.DMA,
        ],
    )
    def _dispatch(x_hbm, pos_hbm, xs_hbm, idx0_v, idx1_v, rows_v, sem):
        wid = lax.axis_index("s") * NC + lax.axis_index("c")
        base = wid * TPW
        pltpu.sync_copy(pos_hbm.at[pl.ds(base, TPW)], idx0_v)
        pltpu.sync_copy(pos_hbm.at[pl.ds(N + base, TPW)], idx1_v)
        pltpu.sync_copy(x_hbm.at[pl.ds(base, TPW)], rows_v)
        cp0 = pltpu.async_copy(rows_v, xs_hbm.at[idx0_v], sem)
        cp1 = pltpu.async_copy(rows_v, xs_hbm.at[idx1_v], sem)
        cp0.wait()
        cp1.wait()

    @functools.partial(
        pl.kernel, mesh=mesh,
        out_type=jax.ShapeDtypeStruct((N, D), _f32),
        scratch_types=[
            pltpu.VMEM((TPW,), _i32),
            pltpu.VMEM((TPW,), _i32),
            pltpu.VMEM((TPW, 16), _f32),
            pltpu.VMEM((TPW, 16), _f32),
            pltpu.VMEM((TPW, D), _f32),
            pltpu.VMEM((TPW, D), _f32),
            pltpu.SemaphoreType.DMA,
        ],
    )
    def _combine(ys_hbm, pos_hbm, w_hbm, y_hbm,
                 idx0_v, idx1_v, w0_v, w1_v, r0_v, r1_v, sem):
        wid = lax.axis_index("s") * NC + lax.axis_index("c")
        base = wid * TPW
        pltpu.sync_copy(pos_hbm.at[pl.ds(base, TPW)], idx0_v)
        pltpu.sync_copy(pos_hbm.at[pl.ds(N + base, TPW)], idx1_v)
        pltpu.sync_copy(w_hbm.at[pl.ds(base, TPW)], w0_v)
        pltpu.sync_copy(w_hbm.at[pl.ds(N + base, TPW)], w1_v)
        cp0 = pltpu.async_copy(ys_hbm.at[idx0_v], r0_v, sem)
        cp1 = pltpu.async_copy(ys_hbm.at[idx1_v], r1_v, sem)
        cp0.wait()
        cp1.wait()

        def row(i, carry):
            w0 = w0_v[i, :]
            w1 = w1_v[i, :]
            for cch in range(D // 16):
                sl = (i, pl.ds(cch * 16, 16))
                r0_v[sl] = r0_v[sl] * w0 + r1_v[sl] * w1
            return carry

        lax.fori_loop(0, TPW, row, 0)
        pltpu.sync_copy(r0_v, y_hbm.at[pl.ds(base, TPW)])

    return _dispatch, _combine


def _ffn_body(te_ref, en_ref, sl_ref, fe_ref, f3_ref, xs_ref,
              w1_hbm, w3_hbm, w2_hbm, ys_ref, w1b, w3b, w2b, s1, s3, s2):
    i = pl.program_id(0)

    def issue(e, slot):
        pltpu.make_async_copy(w1_hbm.at[e], w1b.at[slot], s1.at[slot]).start()
        pltpu.make_async_copy(w3_hbm.at[e], w3b.at[slot], s3.at[slot]).start()
        pltpu.make_async_copy(w2_hbm.at[e], w2b.at[slot], s2.at[slot]).start()

    @pl.when(i == 0)
    def _():
        for p in range(3):
            e0 = f3_ref[p]

            @pl.when(e0 >= 0)
            def _():
                issue(e0, p)

    @pl.when((i > 0) & (fe_ref[i] >= 0))
    def _():
        sd = sl_ref[i] + 2
        sd = jnp.where(sd >= 3, sd - 3, sd)
        issue(fe_ref[i], sd)

    @pl.when(en_ref[i] == 1)
    def _():
        e = te_ref[i]
        slot = sl_ref[i]
        pltpu.make_async_copy(w1_hbm.at[e], w1b.at[slot], s1.at[slot]).wait()
        pltpu.make_async_copy(w3_hbm.at[e], w3b.at[slot], s3.at[slot]).wait()
        pltpu.make_async_copy(w2_hbm.at[e], w2b.at[slot], s2.at[slot]).wait()

    @pl.when(te_ref[i] < E)
    def _():
        slot = sl_ref[i]
        xb = xs_ref[...]
        h1 = lax.dot_general(xb, w1b[slot], (((1,), (1,)), ((), ())),
                             preferred_element_type=_f32)
        h3 = lax.dot_general(xb, w3b[slot], (((1,), (1,)), ((), ())),
                             preferred_element_type=_f32)
        h = h1 * jax.nn.sigmoid(h1) * h3
        ys_ref[...] = lax.dot_general(h, w2b[slot], (((1,), (1,)), ((), ())),
                                      preferred_element_type=_f32)


def _ffn(te, en, sl, fe, f3, xs, W1, W3, W2):
    grid_spec = pltpu.PrefetchScalarGridSpec(
        num_scalar_prefetch=5,
        grid=(G,),
        in_specs=[
            pl.BlockSpec((TM, D), lambda i, *_: (i, 0)),
            pl.BlockSpec(memory_space=pl.ANY),
            pl.BlockSpec(memory_space=pl.ANY),
            pl.BlockSpec(memory_space=pl.ANY),
        ],
        out_specs=pl.BlockSpec((TM, D), lambda i, *_: (i, 0)),
        scratch_shapes=[
            pltpu.VMEM((3, DFF, D), _f32),
            pltpu.VMEM((3, DFF, D), _f32),
            pltpu.VMEM((3, D, DFF), _f32),
            pltpu.SemaphoreType.DMA((3,)),
            pltpu.SemaphoreType.DMA((3,)),
            pltpu.SemaphoreType.DMA((3,)),
        ],
    )
    return pl.pallas_call(
        _ffn_body,
        grid_spec=grid_spec,
        out_shape=jax.ShapeDtypeStruct((CAP, D), _f32),
    )(te, en, sl, fe, f3, xs, W1, W3, W2)


def kernel(x, Wg, W1, W3, W2):
    b, s, d = x.shape
    xf = x.reshape(N, D)
    pos_col, sched, w_bcast = _gate_route(xf, Wg)
    pos_flat = pos_col.reshape(NP)
    te = sched[0, :G]
    en = sched[1, :G]
    sl = sched[2, :G]
    fe = sched[3, :G]
    f3 = sched[4, :3]
    dispatch, combine = _sc_kernels()
    xs = dispatch(xf, pos_flat)
    ys = _ffn(te, en, sl, fe, f3, xs, W1, W3, W2)
    y = combine(ys, pos_flat, w_bcast)
    return y.reshape(b, s, d)


# speedup vs baseline: 1.1928x; 1.1928x over previous
import functools

import jax
import jax.numpy as jnp
from jax import lax
from jax.experimental import pallas as pl
from jax.experimental.pallas import tpu as pltpu
from jax.experimental.pallas import tpu_sc as plsc

E = 8
K = 2
D = 768
DFF = 1024
N = 2048
TM = 512
NP = N * K
G = (NP + E * (TM - 1)) // TM
CAP = G * TM

NC = 2
NS = 16
NW = NC * NS
TPW = N // NW

_f32 = jnp.float32
_i32 = jnp.int32


_NBLK = NP // TM


def _gate_route_body(x_ref, wg_ref, pos_ref, te_ref, w_ref):
    xv = x_ref[...]
    logits = lax.dot_general(xv, wg_ref[...], (((1,), (1,)), ((), ())),
                             preferred_element_type=_f32)
    eio = lax.broadcasted_iota(_i32, (N, E), 1)
    m1 = jnp.max(logits, axis=1, keepdims=True)
    i1 = jnp.min(jnp.where(logits == m1, eio, E), axis=1, keepdims=True)
    masked = jnp.where(eio == i1, -jnp.inf, logits)
    m2 = jnp.max(masked, axis=1, keepdims=True)
    i2 = jnp.min(jnp.where(masked == m2, eio, E), axis=1, keepdims=True)
    w1 = jax.nn.sigmoid(m1 - m2)
    w2 = jax.nn.sigmoid(m2 - m1)
    w_ref[...] = jnp.broadcast_to(jnp.concatenate([w1, w2], axis=0), (NP, 16))

    e_all = jnp.concatenate([i1, i2], axis=0)
    oh_all = (e_all == lax.broadcasted_iota(_i32, (NP, E), 1)).astype(_f32)
    r = lax.broadcasted_iota(_i32, (TM, TM), 0)
    c = lax.broadcasted_iota(_i32, (TM, TM), 1)
    ltri = (c < r).astype(_f32)
    cnt = jnp.zeros((1, E), _f32)
    ranks = []
    for b in range(_NBLK):
        oh = lax.slice(oh_all, (b * TM, 0), ((b + 1) * TM, E))
        local = lax.dot_general(ltri, oh, (((1,), (0,)), ((), ())),
                                preferred_element_type=_f32)
        ranks.append(jnp.sum((local + cnt) * oh, axis=1, keepdims=True))
        cnt = cnt + jnp.sum(oh, axis=0, keepdims=True)
    rank_all = jnp.concatenate(ranks, axis=0)

    padded = jnp.floor((cnt + (TM - 1)) * (1.0 / TM)) * TM
    uio = lax.broadcasted_iota(_i32, (E, E), 0)
    cio = lax.broadcasted_iota(_i32, (E, E), 1)
    utri = (uio < cio).astype(_f32)
    start = lax.dot_general(padded, utri, (((1,), (0,)), ((), ())),
                            preferred_element_type=_f32)
    startsel = jnp.sum(start * oh_all, axis=1, keepdims=True)
    pos_ref[...] = (rank_all + startsel).astype(_i32)

    ivec = lax.broadcasted_iota(_i32, (1, 128), 1)
    te = jnp.full((1, 128), -1, _i32)
    e_last = jnp.zeros((1, 128), _i32)
    for e in range(E):
        s_e = (start[0, e] * (1.0 / TM)).astype(_i32)
        te = te + (ivec >= s_e).astype(_i32)
        has = (cnt[0, e] > 0.0).astype(_i32)
        e_last = jnp.maximum(e_last, has * e)
    n_act = ((start[0, E - 1] + padded[0, E - 1]) * (1.0 / TM)).astype(_i32)
    te_ref[...] = jnp.where(ivec < n_act, te, e_last + E)


def _gate_route(xf, Wg):
    return pl.pallas_call(
        _gate_route_body,
        out_shape=(jax.ShapeDtypeStruct((NP, 1), _i32),
                   jax.ShapeDtypeStruct((1, 128), _i32),
                   jax.ShapeDtypeStruct((NP, 16), _f32)),
    )(xf, Wg)


@functools.cache
def _sc_kernels():
    mesh = plsc.VectorSubcoreMesh(core_axis_name="c", subcore_axis_name="s")

    @functools.partial(
        pl.kernel, mesh=mesh,
        out_type=jax.ShapeDtypeStruct((CAP, D), _f32),
        scratch_types=[
            pltpu.VMEM((TPW,), _i32),
            pltpu.VMEM((TPW,), _i32),
            pltpu.VMEM((TPW, D), _f32),
            pltpu.SemaphoreType.DMA,
        ],
    )
    def _dispatch(x_hbm, pos_hbm, xs_hbm, idx0_v, idx1_v, rows_v, sem):
        wid = lax.axis_index("s") * NC + lax.axis_index("c")
        base = wid * TPW
        pltpu.sync_copy(pos_hbm.at[pl.ds(base, TPW)], idx0_v)
        pltpu.sync_copy(pos_hbm.at[pl.ds(N + base, TPW)], idx1_v)
        pltpu.sync_copy(x_hbm.at[pl.ds(base, TPW)], rows_v)
        cp0 = pltpu.async_copy(rows_v, xs_hbm.at[idx0_v], sem)
        cp1 = pltpu.async_copy(rows_v, xs_hbm.at[idx1_v], sem)
        cp0.wait()
        cp1.wait()

    @functools.partial(
        pl.kernel, mesh=mesh,
        out_type=jax.ShapeDtypeStruct((N, D), _f32),
        scratch_types=[
            pltpu.VMEM((TPW,), _i32),
            pltpu.VMEM((TPW,), _i32),
            pltpu.VMEM((TPW, 16), _f32),
            pltpu.VMEM((TPW, 16), _f32),
            pltpu.VMEM((TPW, D), _f32),
            pltpu.VMEM((TPW, D), _f32),
            pltpu.SemaphoreType.DMA,
        ],
    )
    def _combine(ys_hbm, pos_hbm, w_hbm, y_hbm,
                 idx0_v, idx1_v, w0_v, w1_v, r0_v, r1_v, sem):
        wid = lax.axis_index("s") * NC + lax.axis_index("c")
        base = wid * TPW
        pltpu.sync_copy(pos_hbm.at[pl.ds(base, TPW)], idx0_v)
        pltpu.sync_copy(pos_hbm.at[pl.ds(N + base, TPW)], idx1_v)
        pltpu.sync_copy(w_hbm.at[pl.ds(base, TPW)], w0_v)
        pltpu.sync_copy(w_hbm.at[pl.ds(N + base, TPW)], w1_v)
        cp0 = pltpu.async_copy(ys_hbm.at[idx0_v], r0_v, sem)
        cp1 = pltpu.async_copy(ys_hbm.at[idx1_v], r1_v, sem)
        cp0.wait()
        cp1.wait()

        def row(i, carry):
            w0 = w0_v[i, :]
            w1 = w1_v[i, :]
            for cch in range(D // 16):
                sl = (i, pl.ds(cch * 16, 16))
                r0_v[sl] = r0_v[sl] * w0 + r1_v[sl] * w1
            return carry

        lax.fori_loop(0, TPW, row, 0)
        pltpu.sync_copy(r0_v, y_hbm.at[pl.ds(base, TPW)])

    return _dispatch, _combine


def _ffn_body(te_ref, xs_ref, w1_ref, w3_ref, w2_ref, ys_ref):
    i = pl.program_id(0)

    @pl.when(te_ref[i] < E)
    def _():
        xb = xs_ref[...]
        h1 = lax.dot_general(xb, w1_ref[0], (((1,), (1,)), ((), ())),
                             preferred_element_type=_f32)
        h3 = lax.dot_general(xb, w3_ref[0], (((1,), (1,)), ((), ())),
                             preferred_element_type=_f32)
        h = h1 * jax.nn.sigmoid(h1) * h3
        ys_ref[...] = lax.dot_general(h, w2_ref[0], (((1,), (1,)), ((), ())),
                                      preferred_element_type=_f32)


def _wmap(i, te_ref):
    return (te_ref[i] % E, 0, 0)


def _ffn(te, xs, W1, W3, W2):
    grid_spec = pltpu.PrefetchScalarGridSpec(
        num_scalar_prefetch=1,
        grid=(G,),
        in_specs=[
            pl.BlockSpec((TM, D), lambda i, te_ref: (i, 0)),
            pl.BlockSpec((1, DFF, D), _wmap),
            pl.BlockSpec((1, DFF, D), _wmap),
            pl.BlockSpec((1, D, DFF), _wmap),
        ],
        out_specs=pl.BlockSpec((TM, D), lambda i, te_ref: (i, 0)),
    )
    return pl.pallas_call(
        _ffn_body,
        grid_spec=grid_spec,
        out_shape=jax.ShapeDtypeStruct((CAP, D), _f32),
    )(te, xs, W1, W3, W2)


def kernel(x, Wg, W1, W3, W2):
    b, s, d = x.shape
    xf = x.reshape(N, D)
    pos_col, te_pad, w_bcast = _gate_route(xf, Wg)
    pos_flat = pos_col.reshape(NP)
    te = te_pad.reshape(128)[:G]
    dispatch, combine = _sc_kernels()
    xs = dispatch(xf, pos_flat)
    ys = _ffn(te, xs, W1, W3, W2)
    y = combine(ys, pos_flat, w_bcast)
    return y.reshape(b, s, d)


# speedup vs baseline: 1.2077x; 1.0125x over previous
import functools

import jax
import jax.numpy as jnp
from jax import lax
from jax.experimental import pallas as pl
from jax.experimental.pallas import tpu as pltpu
from jax.experimental.pallas import tpu_sc as plsc

E = 8
K = 2
D = 768
DFF = 1024
N = 2048
TM = 512
NP = N * K
G = (NP + E * (TM - 1)) // TM
CAP = G * TM

NC = 2
NS = 16
NW = NC * NS
TPW = N // NW

_f32 = jnp.float32
_i32 = jnp.int32


_NBLK = NP // TM


def _gate_route_body(x_ref, wg_ref, pos_ref, te_ref, w_ref):
    xv = x_ref[...]
    logits = lax.dot_general(xv, wg_ref[...], (((1,), (1,)), ((), ())),
                             preferred_element_type=_f32)
    eio = lax.broadcasted_iota(_i32, (N, E), 1)
    m1 = jnp.max(logits, axis=1, keepdims=True)
    i1 = jnp.min(jnp.where(logits == m1, eio, E), axis=1, keepdims=True)
    masked = jnp.where(eio == i1, -jnp.inf, logits)
    m2 = jnp.max(masked, axis=1, keepdims=True)
    i2 = jnp.min(jnp.where(masked == m2, eio, E), axis=1, keepdims=True)
    w1 = jax.nn.sigmoid(m1 - m2)
    w2 = jax.nn.sigmoid(m2 - m1)
    w_ref[...] = jnp.broadcast_to(jnp.concatenate([w1, w2], axis=0), (NP, 16))

    e_all = jnp.concatenate([i1, i2], axis=0)
    oh_all = (e_all == lax.broadcasted_iota(_i32, (NP, E), 1)).astype(_f32)
    r = lax.broadcasted_iota(_i32, (TM, TM), 0)
    c = lax.broadcasted_iota(_i32, (TM, TM), 1)
    ltri = (c < r).astype(_f32)
    cnt = jnp.zeros((1, E), _f32)
    ranks = []
    for b in range(_NBLK):
        oh = lax.slice(oh_all, (b * TM, 0), ((b + 1) * TM, E))
        local = lax.dot_general(ltri, oh, (((1,), (0,)), ((), ())),
                                preferred_element_type=_f32)
        ranks.append(jnp.sum((local + cnt) * oh, axis=1, keepdims=True))
        cnt = cnt + jnp.sum(oh, axis=0, keepdims=True)
    rank_all = jnp.concatenate(ranks, axis=0)

    padded = jnp.floor((cnt + (TM - 1)) * (1.0 / TM)) * TM
    uio = lax.broadcasted_iota(_i32, (E, E), 0)
    cio = lax.broadcasted_iota(_i32, (E, E), 1)
    utri = (uio < cio).astype(_f32)
    start = lax.dot_general(padded, utri, (((1,), (0,)), ((), ())),
                            preferred_element_type=_f32)
    startsel = jnp.sum(start * oh_all, axis=1, keepdims=True)
    pos_ref[...] = (rank_all + startsel).astype(_i32)

    ivec = lax.broadcasted_iota(_i32, (1, 128), 1)
    te = jnp.full((1, 128), -1, _i32)
    e_last = jnp.zeros((1, 128), _i32)
    for e in range(E):
        s_e = (start[0, e] * (1.0 / TM)).astype(_i32)
        te = te + (ivec >= s_e).astype(_i32)
        has = (cnt[0, e] > 0.0).astype(_i32)
        e_last = jnp.maximum(e_last, has * e)
    n_act = ((start[0, E - 1] + padded[0, E - 1]) * (1.0 / TM)).astype(_i32)
    act = ivec < n_act
    te_row = jnp.where(act, te, e_last + E)
    xi_row = jnp.where(act, ivec, n_act - 1)
    oi_row = jnp.where(act, ivec, G - 1)
    te_ref[...] = jnp.concatenate(
        [te_row, xi_row, oi_row, jnp.zeros((5, 128), _i32)], axis=0)


def _gate_route(xf, Wg):
    return pl.pallas_call(
        _gate_route_body,
        out_shape=(jax.ShapeDtypeStruct((NP, 1), _i32),
                   jax.ShapeDtypeStruct((8, 128), _i32),
                   jax.ShapeDtypeStruct((NP, 16), _f32)),
    )(xf, Wg)


@functools.cache
def _sc_kernels():
    mesh = plsc.VectorSubcoreMesh(core_axis_name="c", subcore_axis_name="s")

    @functools.partial(
        pl.kernel, mesh=mesh,
        out_type=jax.ShapeDtypeStruct((CAP, D), _f32),
        scratch_types=[
            pltpu.VMEM((TPW,), _i32),
            pltpu.VMEM((TPW,), _i32),
            pltpu.VMEM((TPW, D), _f32),
            pltpu.SemaphoreType.DMA,
        ],
    )
    def _dispatch(x_hbm, pos_hbm, xs_hbm, idx0_v, idx1_v, rows_v, sem):
        wid = lax.axis_index("s") * NC + lax.axis_index("c")
        base = wid * TPW
        pltpu.sync_copy(pos_hbm.at[pl.ds(base, TPW)], idx0_v)
        pltpu.sync_copy(pos_hbm.at[pl.ds(N + base, TPW)], idx1_v)
        pltpu.sync_copy(x_hbm.at[pl.ds(base, TPW)], rows_v)
        cp0 = pltpu.async_copy(rows_v, xs_hbm.at[idx0_v], sem)
        cp1 = pltpu.async_copy(rows_v, xs_hbm.at[idx1_v], sem)
        cp0.wait()
        cp1.wait()

    @functools.partial(
        pl.kernel, mesh=mesh,
        out_type=jax.ShapeDtypeStruct((N, D), _f32),
        scratch_types=[
            pltpu.VMEM((TPW,), _i32),
            pltpu.VMEM((TPW,), _i32),
            pltpu.VMEM((TPW, 16), _f32),
            pltpu.VMEM((TPW, 16), _f32),
            pltpu.VMEM((TPW, D), _f32),
            pltpu.VMEM((TPW, D), _f32),
            pltpu.SemaphoreType.DMA,
        ],
    )
    def _combine(ys_hbm, pos_hbm, w_hbm, y_hbm,
                 idx0_v, idx1_v, w0_v, w1_v, r0_v, r1_v, sem):
        wid = lax.axis_index("s") * NC + lax.axis_index("c")
        base = wid * TPW
        pltpu.sync_copy(pos_hbm.at[pl.ds(base, TPW)], idx0_v)
        pltpu.sync_copy(pos_hbm.at[pl.ds(N + base, TPW)], idx1_v)
        pltpu.sync_copy(w_hbm.at[pl.ds(base, TPW)], w0_v)
        pltpu.sync_copy(w_hbm.at[pl.ds(N + base, TPW)], w1_v)
        cp0 = pltpu.async_copy(ys_hbm.at[idx0_v], r0_v, sem)
        cp1 = pltpu.async_copy(ys_hbm.at[idx1_v], r1_v, sem)
        cp0.wait()
        cp1.wait()

        def row(i, carry):
            w0 = w0_v[i, :]
            w1 = w1_v[i, :]
            for cch in range(D // 16):
                sl = (i, pl.ds(cch * 16, 16))
                r0_v[sl] = r0_v[sl] * w0 + r1_v[sl] * w1
            return carry

        lax.fori_loop(0, TPW, row, 0)
        pltpu.sync_copy(r0_v, y_hbm.at[pl.ds(base, TPW)])

    return _dispatch, _combine


def _ffn_body(te_ref, xi_ref, oi_ref, xs_ref, w1_ref, w3_ref, w2_ref, ys_ref):
    i = pl.program_id(0)

    @pl.when(te_ref[i] < E)
    def _():
        xb = xs_ref[...]
        h1 = lax.dot_general(xb, w1_ref[0], (((1,), (1,)), ((), ())),
                             preferred_element_type=_f32)
        h3 = lax.dot_general(xb, w3_ref[0], (((1,), (1,)), ((), ())),
                             preferred_element_type=_f32)
        h = h1 * jax.nn.sigmoid(h1) * h3
        ys_ref[...] = lax.dot_general(h, w2_ref[0], (((1,), (1,)), ((), ())),
                                      preferred_element_type=_f32)


def _wmap(i, te_ref, xi_ref, oi_ref):
    return (te_ref[i] % E, 0, 0)


def _ffn(te, xi, oi, xs, W1, W3, W2):
    grid_spec = pltpu.PrefetchScalarGridSpec(
        num_scalar_prefetch=3,
        grid=(G,),
        in_specs=[
            pl.BlockSpec((TM, D), lambda i, te_ref, xi_ref, oi_ref:
                         (xi_ref[i], 0)),
            pl.BlockSpec((1, DFF, D), _wmap),
            pl.BlockSpec((1, DFF, D), _wmap),
            pl.BlockSpec((1, D, DFF), _wmap),
        ],
        out_specs=pl.BlockSpec((TM, D), lambda i, te_ref, xi_ref, oi_ref:
                               (oi_ref[i], 0)),
    )
    return pl.pallas_call(
        _ffn_body,
        grid_spec=grid_spec,
        out_shape=jax.ShapeDtypeStruct((CAP, D), _f32),
    )(te, xi, oi, xs, W1, W3, W2)


def kernel(x, Wg, W1, W3, W2):
    b, s, d = x.shape
    xf = x.reshape(N, D)
    pos_col, sched, w_bcast = _gate_route(xf, Wg)
    pos_flat = pos_col.reshape(NP)
    te = sched[0, :G]
    xi = sched[1, :G]
    oi = sched[2, :G]
    dispatch, combine = _sc_kernels()
    xs = dispatch(xf, pos_flat)
    ys = _ffn(te, xi, oi, xs, W1, W3, W2)
    y = combine(ys, pos_flat, w_bcast)
    return y.reshape(b, s, d)
